# baseline (speedup 1.0000x reference)
import functools

import jax
import jax.numpy as jnp
from jax import lax
from jax.experimental import pallas as pl
from jax.experimental.pallas import tpu as pltpu
from jax.experimental.pallas import tpu_sc as plsc

B = 16384
EMB = 16
HID = 32
N_USERS = 1000000
N_QUIZZES = 100000
ICH = 128


def _make_gather():
    info = plsc.get_sparse_core_info()
    nw = info.num_cores * info.num_subcores
    b_per_w = B // nw
    n_gath = b_per_w * EMB
    n_tr = n_gath // ICH
    mesh = plsc.VectorSubcoreMesh(core_axis_name="c", subcore_axis_name="s")

    @functools.partial(
        pl.kernel,
        mesh=mesh,
        out_type=jax.ShapeDtypeStruct((2 * EMB, B), jnp.float32),
        scratch_types=[
            pltpu.VMEM((b_per_w,), jnp.int32),
            pltpu.VMEM((b_per_w,), jnp.int32),
            pltpu.VMEM((n_gath,), jnp.int32),
            pltpu.VMEM((n_gath,), jnp.int32),
            pltpu.VMEM((n_gath,), jnp.float32),
            pltpu.VMEM((n_gath,), jnp.float32),
            pltpu.SemaphoreType.DMA,
        ],
        compiler_params=pltpu.CompilerParams(
            needs_layout_passes=False, use_tc_tiling_on_sc=True),
    )
    def gather(uidx_hbm, qidx_hbm, utab_hbm, qtab_hbm, x_hbm,
               uidx_v, qidx_v, uflat_v, qflat_v, ug_v, qg_v, sem):
        wid = lax.axis_index("s") * info.num_cores + lax.axis_index("c")
        base = wid * b_per_w
        pltpu.sync_copy(uidx_hbm.at[pl.ds(base, b_per_w)], uidx_v)
        pltpu.sync_copy(qidx_hbm.at[pl.ds(base, b_per_w)], qidx_v)
        for e in range(EMB):
            for k in range(b_per_w // 16):
                s = pl.ds(k * 16, 16)
                d = pl.ds(e * b_per_w + k * 16, 16)
                uflat_v[d] = uidx_v[s] + (e * N_USERS)
                qflat_v[d] = qidx_v[s] + (e * N_QUIZZES)
        copies = []
        for j in range(n_tr):
            t = pl.ds(j * ICH, ICH)
            copies.append(pltpu.async_copy(
                utab_hbm.at[uflat_v.at[t]], ug_v.at[t], sem))
            copies.append(pltpu.async_copy(
                qtab_hbm.at[qflat_v.at[t]], qg_v.at[t], sem))
        for c in copies:
            c.wait()
        for e in range(EMB):
            s = pl.ds(e * b_per_w, b_per_w)
            pltpu.sync_copy(ug_v.at[s], x_hbm.at[e, pl.ds(base, b_per_w)])
            pltpu.sync_copy(qg_v.at[s],
                            x_hbm.at[EMB + e, pl.ds(base, b_per_w)])

    return gather


_gather = _make_gather()


def _mlp_body(x_ref, t_ref, w1uq_ref, w1t_ref, b1_ref, w2_ref, b2_ref, o_ref):
    h = (jnp.dot(w1uq_ref[...], x_ref[...], preferred_element_type=jnp.float32)
         + jnp.dot(w1t_ref[...], t_ref[...], preferred_element_type=jnp.float32)
         + b1_ref[...])
    h = jnp.maximum(h, 0.0)
    o = jnp.dot(w2_ref[...], h, preferred_element_type=jnp.float32) + b2_ref[...]
    o_ref[...] = jax.nn.sigmoid(o)


_mlp = pl.pallas_call(
    _mlp_body,
    out_shape=jax.ShapeDtypeStruct((1, B), jnp.float32),
)


def kernel(user, quiz, time, user_table, quiz_table, W1, b1, W2, b2):
    x = _gather(user.astype(jnp.int32), quiz.astype(jnp.int32),
                user_table.T.reshape(N_USERS * EMB),
                quiz_table.T.reshape(N_QUIZZES * EMB))
    out = _mlp(x, time.T,
               W1[:2 * EMB].T, W1[2 * EMB:].T,
               b1.reshape(HID, 1), W2.T, b2.reshape(1, 1))
    return out.reshape(B)


# speedup vs baseline: 1.3368x; 1.3368x over previous
import functools

import jax
import jax.numpy as jnp
from jax import lax
from jax.experimental import pallas as pl
from jax.experimental.pallas import tpu as pltpu
from jax.experimental.pallas import tpu_sc as plsc

B = 16384
EMB = 16
HID = 32
N_USERS = 1000000
N_QUIZZES = 100000
ICH = 128


def _make_gather():
    info = plsc.get_sparse_core_info()
    nw = info.num_cores * info.num_subcores
    b_per_w = B // nw
    n_gath = b_per_w * EMB
    n_tr = n_gath // ICH
    mesh = plsc.VectorSubcoreMesh(core_axis_name="c", subcore_axis_name="s")

    @functools.partial(
        pl.kernel,
        mesh=mesh,
        out_type=jax.ShapeDtypeStruct((2 * EMB, B), jnp.float32),
        scratch_types=[
            pltpu.VMEM((b_per_w,), jnp.int32),
            pltpu.VMEM((b_per_w,), jnp.int32),
            pltpu.VMEM((n_gath,), jnp.int32),
            pltpu.VMEM((n_gath,), jnp.int32),
            pltpu.VMEM((n_gath,), jnp.float32),
            pltpu.VMEM((n_gath,), jnp.float32),
            pltpu.SemaphoreType.DMA,
        ],
        compiler_params=pltpu.CompilerParams(
            needs_layout_passes=False, use_tc_tiling_on_sc=True),
    )
    def gather(uidx_hbm, qidx_hbm, utab_hbm, qtab_hbm, x_hbm,
               uidx_v, qidx_v, uflat_v, qflat_v, ug_v, qg_v, sem):
        wid = lax.axis_index("s") * info.num_cores + lax.axis_index("c")
        base = wid * b_per_w
        pltpu.sync_copy(uidx_hbm.at[pl.ds(base, b_per_w)], uidx_v)
        pltpu.sync_copy(qidx_hbm.at[pl.ds(base, b_per_w)], qidx_v)
        for e in range(EMB):
            for k in range(b_per_w // 16):
                s = pl.ds(k * 16, 16)
                d = pl.ds(e * b_per_w + k * 16, 16)
                uflat_v[d] = uidx_v[s] + (e * N_USERS)
                qflat_v[d] = qidx_v[s] + (e * N_QUIZZES)
        copies = []
        for j in range(n_tr):
            t = pl.ds(j * ICH, ICH)
            copies.append(pltpu.async_copy(
                utab_hbm.at[uflat_v.at[t]], ug_v.at[t], sem))
            copies.append(pltpu.async_copy(
                qtab_hbm.at[qflat_v.at[t]], qg_v.at[t], sem))
        for c in copies:
            c.wait()
        for e in range(EMB):
            s = pl.ds(e * b_per_w, b_per_w)
            pltpu.sync_copy(ug_v.at[s], x_hbm.at[e, pl.ds(base, b_per_w)])
            pltpu.sync_copy(qg_v.at[s],
                            x_hbm.at[EMB + e, pl.ds(base, b_per_w)])

    return gather


_gather = _make_gather()


def _mlp_body(x_ref, t_ref, w1uq_ref, w1t_ref, b1_ref, w2_ref, b2_ref, o_ref):
    h = (jnp.dot(w1uq_ref[...], x_ref[...], preferred_element_type=jnp.float32)
         + jnp.dot(w1t_ref[...], t_ref[...], preferred_element_type=jnp.float32)
         + b1_ref[...])
    h = jnp.maximum(h, 0.0)
    o = jnp.dot(w2_ref[...], h, preferred_element_type=jnp.float32) + b2_ref[...]
    o_ref[...] = jax.nn.sigmoid(o)


_mlp = pl.pallas_call(
    _mlp_body,
    out_shape=jax.ShapeDtypeStruct((1, B), jnp.float32),
)


def kernel(user, quiz, time, user_table, quiz_table, W1, b1, W2, b2):
    x = _gather(user.astype(jnp.int32), quiz.astype(jnp.int32),
                jnp.concatenate([user_table[:, e] for e in range(EMB)]),
                jnp.concatenate([quiz_table[:, e] for e in range(EMB)]))
    out = _mlp(x, time.T,
               W1[:2 * EMB].T, W1[2 * EMB:].T,
               b1.reshape(HID, 1), W2.T, b2.reshape(1, 1))
    return out.reshape(B)


# speedup vs baseline: 10.5942x; 7.9251x over previous
import functools

import jax
import jax.numpy as jnp
from jax import lax
from jax.experimental import pallas as pl
from jax.experimental.pallas import tpu as pltpu
from jax.experimental.pallas import tpu_sc as plsc

B = 16384
EMB = 16
HID = 32
N_USERS = 1000000
N_QUIZZES = 100000
LANE = 128
SLAB = 2048
ICH = 128

NB_U = -(-N_USERS // LANE)
NB_Q = -(-N_QUIZZES // LANE)
U1DLEN = NB_U * EMB * LANE
Q1DLEN = NB_Q * EMB * LANE

U_FULL = N_USERS // LANE
Q_FULL = N_QUIZZES // LANE
U_SLABS = U_FULL // (SLAB // LANE)
Q_SLABS = Q_FULL // (SLAB // LANE)
N_SLABS = U_SLABS + Q_SLABS
U_LEFT = U_FULL % (SLAB // LANE)
Q_LEFT = Q_FULL % (SLAB // LANE)
U_TAIL = N_USERS - U_FULL * LANE
Q_TAIL = N_QUIZZES - Q_FULL * LANE


def _make_detile():
    info = plsc.get_sparse_core_info()
    nw = info.num_cores * info.num_subcores
    n_iter = -(-N_SLABS // nw)
    mesh = plsc.VectorSubcoreMesh(core_axis_name="c", subcore_axis_name="s")

    @functools.partial(
        pl.kernel,
        mesh=mesh,
        out_type=[
            jax.ShapeDtypeStruct((NB_U * EMB, LANE), jnp.float32),
            jax.ShapeDtypeStruct((NB_Q * EMB, LANE), jnp.float32),
        ],
        scratch_types=[
            pltpu.VMEM((EMB, SLAB), jnp.float32),
            pltpu.VMEM((EMB, U_TAIL), jnp.float32),
            pltpu.VMEM((EMB, Q_TAIL), jnp.float32),
            pltpu.VMEM((EMB, LANE), jnp.float32),
            pltpu.VMEM((EMB,), jnp.int32),
            pltpu.SemaphoreType.DMA,
        ],
        compiler_params=pltpu.CompilerParams(
            needs_layout_passes=False, use_tc_tiling_on_sc=True),
    )
    def detile(utab_hbm, qtab_hbm, u2d, q2d,
               buf, ubtail, qbtail, stage, ridx, sem):
        wid = lax.axis_index("s") * info.num_cores + lax.axis_index("c")

        def do_slab(tab, out2d, blk0):
            pltpu.sync_copy(tab.at[:, pl.ds(blk0 * LANE, SLAB)], buf)
            outs = []
            for bb in range(SLAB // LANE):
                outs.append(pltpu.async_copy(
                    buf.at[:, pl.ds(bb * LANE, LANE)],
                    out2d.at[pl.ds((blk0 + bb) * EMB, EMB), :], sem))
            for c in outs:
                c.wait()

        for k in range(n_iter):
            s = wid + nw * k

            @pl.when(s < U_SLABS)
            def _():
                do_slab(utab_hbm, u2d, s * (SLAB // LANE))

            @pl.when(jnp.logical_and(s >= U_SLABS, s < N_SLABS))
            def _():
                do_slab(qtab_hbm, q2d, (s - U_SLABS) * (SLAB // LANE))

        @pl.when(wid == 0)
        def _():
            pltpu.sync_copy(
                utab_hbm.at[:, pl.ds(U_SLABS * SLAB, U_LEFT * LANE)],
                buf.at[:, pl.ds(0, U_LEFT * LANE)])
            outs = []
            for bb in range(U_LEFT):
                outs.append(pltpu.async_copy(
                    buf.at[:, pl.ds(bb * LANE, LANE)],
                    u2d.at[pl.ds((U_SLABS * (SLAB // LANE) + bb) * EMB,
                                 EMB), :], sem))
            for c in outs:
                c.wait()

        @pl.when(wid == 1)
        def _():
            pltpu.sync_copy(
                qtab_hbm.at[:, pl.ds(Q_SLABS * SLAB, Q_LEFT * LANE)],
                buf.at[:, pl.ds(0, Q_LEFT * LANE)])
            outs = []
            for bb in range(Q_LEFT):
                outs.append(pltpu.async_copy(
                    buf.at[:, pl.ds(bb * LANE, LANE)],
                    q2d.at[pl.ds((Q_SLABS * (SLAB // LANE) + bb) * EMB,
                                 EMB), :], sem))
            for c in outs:
                c.wait()

        @pl.when(wid == 2)
        def _():
            pltpu.sync_copy(utab_hbm.at[:, pl.ds(U_FULL * LANE, U_TAIL)],
                            ubtail)
            for e in range(EMB):
                for kk in range(U_TAIL // 16):
                    stage[e, pl.ds(kk * 16, 16)] = (
                        ubtail[e, pl.ds(kk * 16, 16)])
            ridx[...] = lax.iota(jnp.int32, EMB) + (U_FULL * EMB)
            pltpu.async_copy(stage, u2d.at[ridx], sem).wait()

        @pl.when(wid == 3)
        def _():
            pltpu.sync_copy(qtab_hbm.at[:, pl.ds(Q_FULL * LANE, Q_TAIL)],
                            qbtail)
            for e in range(EMB):
                for kk in range(Q_TAIL // 16):
                    stage[e, pl.ds(kk * 16, 16)] = (
                        qbtail[e, pl.ds(kk * 16, 16)])
            ridx[...] = lax.iota(jnp.int32, EMB) + (Q_FULL * EMB)
            pltpu.async_copy(stage, q2d.at[ridx], sem).wait()

    return detile


def _make_gather():
    info = plsc.get_sparse_core_info()
    nw = info.num_cores * info.num_subcores
    b_per_w = B // nw
    n_gath = b_per_w * EMB
    n_tr = n_gath // ICH
    mesh = plsc.VectorSubcoreMesh(core_axis_name="c", subcore_axis_name="s")

    @functools.partial(
        pl.kernel,
        mesh=mesh,
        out_type=jax.ShapeDtypeStruct((2 * EMB, B), jnp.float32),
        scratch_types=[
            pltpu.VMEM((b_per_w,), jnp.int32),
            pltpu.VMEM((b_per_w,), jnp.int32),
            pltpu.VMEM((n_gath,), jnp.int32),
            pltpu.VMEM((n_gath,), jnp.int32),
            pltpu.VMEM((n_gath,), jnp.float32),
            pltpu.VMEM((n_gath,), jnp.float32),
            pltpu.SemaphoreType.DMA,
        ],
        compiler_params=pltpu.CompilerParams(
            needs_layout_passes=False, use_tc_tiling_on_sc=True),
    )
    def gather(uidx_hbm, qidx_hbm, utab_hbm, qtab_hbm, x_hbm,
               uidx_v, qidx_v, uflat_v, qflat_v, ug_v, qg_v, sem):
        wid = lax.axis_index("s") * info.num_cores + lax.axis_index("c")
        base = wid * b_per_w
        pltpu.sync_copy(uidx_hbm.at[pl.ds(base, b_per_w)], uidx_v)
        pltpu.sync_copy(qidx_hbm.at[pl.ds(base, b_per_w)], qidx_v)
        for e in range(EMB):
            for k in range(b_per_w // 16):
                s = pl.ds(k * 16, 16)
                d = pl.ds(e * b_per_w + k * 16, 16)
                uv = uidx_v[s]
                qv = qidx_v[s]
                uflat_v[d] = (lax.shift_right_logical(uv, 7) * (EMB * LANE)
                              + (uv & (LANE - 1)) + (e * LANE))
                qflat_v[d] = (lax.shift_right_logical(qv, 7) * (EMB * LANE)
                              + (qv & (LANE - 1)) + (e * LANE))
        copies = []
        for j in range(n_tr):
            t = pl.ds(j * ICH, ICH)
            copies.append(pltpu.async_copy(
                utab_hbm.at[uflat_v.at[t]], ug_v.at[t], sem))
            copies.append(pltpu.async_copy(
                qtab_hbm.at[qflat_v.at[t]], qg_v.at[t], sem))
        for c in copies:
            c.wait()
        for e in range(EMB):
            s = pl.ds(e * b_per_w, b_per_w)
            pltpu.sync_copy(ug_v.at[s], x_hbm.at[e, pl.ds(base, b_per_w)])
            pltpu.sync_copy(qg_v.at[s],
                            x_hbm.at[EMB + e, pl.ds(base, b_per_w)])

    return gather


_detile = _make_detile()
_gather = _make_gather()


def _mlp_body(x_ref, t_ref, w1uq_ref, w1t_ref, b1_ref, w2_ref, b2_ref, o_ref):
    h = (jnp.dot(w1uq_ref[...], x_ref[...], preferred_element_type=jnp.float32)
         + jnp.dot(w1t_ref[...], t_ref[...], preferred_element_type=jnp.float32)
         + b1_ref[...])
    h = jnp.maximum(h, 0.0)
    o = jnp.dot(w2_ref[...], h, preferred_element_type=jnp.float32) + b2_ref[...]
    o_ref[...] = jax.nn.sigmoid(o)


_mlp = pl.pallas_call(
    _mlp_body,
    out_shape=jax.ShapeDtypeStruct((1, B), jnp.float32),
)


def kernel(user, quiz, time, user_table, quiz_table, W1, b1, W2, b2):
    u2d, q2d = _detile(user_table.T, quiz_table.T)
    x = _gather(user.astype(jnp.int32), quiz.astype(jnp.int32),
                u2d.reshape(U1DLEN), q2d.reshape(Q1DLEN))
    out = _mlp(x, time.T,
               W1[:2 * EMB].T, W1[2 * EMB:].T,
               b1.reshape(HID, 1), W2.T, b2.reshape(1, 1))
    return out.reshape(B)
